# baseline (speedup 1.0000x reference)
import functools

import jax
import jax.numpy as jnp
from jax import lax
from jax.experimental import pallas as pl
from jax.experimental.pallas import tpu as pltpu
from jax.experimental.pallas import tpu_sc as plsc

_N = 10000
_NP = 10240
_E = 320000
_D_IN, _HID, _D_OUT = 128, 64, 32
_T = 128

_NC, _NS = 2, 16
_NW = _NC * _NS
_CH = 128
_K = 79
_EP = _NW * _K * _CH
_RPS = _NP // _NS

_BLK = 256
_HI = jax.lax.Precision.HIGHEST


def _mm1_body(x_ref, w_ref, tab_ref, r_ref):
    m = jnp.dot(x_ref[...], w_ref[...], preferred_element_type=jnp.float32,
                precision=_HI)
    lane = lax.broadcasted_iota(jnp.int32, (_BLK, _T - _HID), 1)
    ones = jnp.where(lane == 0, 1.0, 0.0).astype(jnp.float32)
    tab_ref[...] = jnp.concatenate([m[:, :_HID], ones], axis=1)
    r_ref[...] = m[:, _HID:]


def _mid_body(acc_ref, r1_ref, b1_ref, w2_ref, tab2_ref, r2_ref, ci_ref):
    acc = acc_ref[0] + acc_ref[1]
    cnt_inv = 1.0 / jnp.maximum(acc[:, _HID:_HID + 1], 1.0)
    h = jnp.maximum(acc[:, :_HID] * cnt_inv + b1_ref[0] + r1_ref[...], 0.0)
    m = jnp.dot(h, w2_ref[...], preferred_element_type=jnp.float32,
                precision=_HI)
    zeros = jnp.zeros((_BLK, _T - _D_OUT), jnp.float32)
    tab2_ref[...] = jnp.concatenate([m[:, :_D_OUT], zeros], axis=1)
    r2_ref[...] = m[:, _D_OUT:]
    ci_ref[...] = jnp.broadcast_to(cnt_inv, (_BLK, 8))


def _out_body(acc_ref, r2_ref, b2_ref, ci_ref, o_ref):
    acc = acc_ref[0, :, :_D_OUT] + acc_ref[1, :, :_D_OUT]
    o_ref[...] = jnp.maximum(acc * ci_ref[:, :1] + b2_ref[0] + r2_ref[...],
                             0.0)


def _sc_segsum(table, eidx, zeros):
    mesh = plsc.VectorSubcoreMesh(core_axis_name="c", subcore_axis_name="s")

    @functools.partial(
        pl.kernel, mesh=mesh,
        out_type=jax.ShapeDtypeStruct((_NC, _NP, _T), jnp.float32),
        scratch_types=[
            pltpu.VMEM_SHARED((_NP, _T), jnp.float32),
        ],
    )
    def k(tab_hbm, eidx_hbm, z_hbm, out_hbm, acc_sh):
        cid = lax.axis_index("c")
        sid = lax.axis_index("s")
        wid = sid * _NC + cid
        row0 = sid * _RPS
        pltpu.sync_copy(z_hbm.at[pl.ds(row0, _RPS)],
                        acc_sh.at[pl.ds(row0, _RPS)])
        plsc.subcore_barrier()

        def body(idx_v, buf):
            @pl.loop(0, _K)
            def _(j):
                pltpu.sync_copy(eidx_hbm.at[wid, j], idx_v)
                pltpu.sync_copy(tab_hbm.at[idx_v.at[0]], buf)
                pltpu.sync_copy(buf, acc_sh.at[idx_v.at[1]], add=True)

        pl.run_scoped(body,
                      pltpu.VMEM((2, _CH), jnp.int32),
                      pltpu.VMEM((_CH, _T), jnp.float32))
        plsc.subcore_barrier()
        pltpu.sync_copy(acc_sh.at[pl.ds(row0, _RPS)],
                        out_hbm.at[cid, pl.ds(row0, _RPS)])

    return k(table, eidx, zeros)


def kernel(x, edge_index, Wl1, bl1, Wr1, Wl2, bl2, Wr2):
    xp = jnp.pad(x, ((0, _NP - _N), (0, 0)))
    w1c = jnp.concatenate([Wl1.T, Wr1.T], axis=1)
    w2c = jnp.concatenate([Wl2.T, Wr2.T], axis=1)
    b1 = bl1.reshape(1, _HID)
    b2 = bl2.reshape(1, _D_OUT)
    pad_idx = (_N + jnp.arange(_EP - _E, dtype=jnp.int32) % (_NP - _N))
    pad_idx = jnp.broadcast_to(pad_idx, (2, _EP - _E))
    eidx = jnp.concatenate([edge_index, pad_idx], axis=1)
    eidx = eidx.reshape(2, _NW, _K, _CH)
    eidx = jnp.transpose(eidx, (1, 2, 0, 3))
    z = jnp.zeros((_NP, _T), jnp.float32)

    tab1, r1 = pl.pallas_call(
        _mm1_body,
        grid=(_NP // _BLK,),
        in_specs=[pl.BlockSpec((_BLK, _D_IN), lambda i: (i, 0)),
                  pl.BlockSpec((_D_IN, 2 * _HID), lambda i: (0, 0))],
        out_specs=[pl.BlockSpec((_BLK, _T), lambda i: (i, 0)),
                   pl.BlockSpec((_BLK, _HID), lambda i: (i, 0))],
        out_shape=[jax.ShapeDtypeStruct((_NP, _T), jnp.float32),
                   jax.ShapeDtypeStruct((_NP, _HID), jnp.float32)],
    )(xp, w1c)

    acc1 = _sc_segsum(tab1, eidx, z)

    tab2, r2, ci = pl.pallas_call(
        _mid_body,
        grid=(_NP // _BLK,),
        in_specs=[pl.BlockSpec((_NC, _BLK, _T), lambda i: (0, i, 0)),
                  pl.BlockSpec((_BLK, _HID), lambda i: (i, 0)),
                  pl.BlockSpec((1, _HID), lambda i: (0, 0)),
                  pl.BlockSpec((_HID, 2 * _D_OUT), lambda i: (0, 0))],
        out_specs=[pl.BlockSpec((_BLK, _T), lambda i: (i, 0)),
                   pl.BlockSpec((_BLK, _D_OUT), lambda i: (i, 0)),
                   pl.BlockSpec((_BLK, 8), lambda i: (i, 0))],
        out_shape=[jax.ShapeDtypeStruct((_NP, _T), jnp.float32),
                   jax.ShapeDtypeStruct((_NP, _D_OUT), jnp.float32),
                   jax.ShapeDtypeStruct((_NP, 8), jnp.float32)],
    )(acc1, r1, b1, w2c)

    acc2 = _sc_segsum(tab2, eidx, z)

    out = pl.pallas_call(
        _out_body,
        grid=(_NP // _BLK,),
        in_specs=[pl.BlockSpec((_NC, _BLK, _T), lambda i: (0, i, 0)),
                  pl.BlockSpec((_BLK, _D_OUT), lambda i: (i, 0)),
                  pl.BlockSpec((1, _D_OUT), lambda i: (0, 0)),
                  pl.BlockSpec((_BLK, 8), lambda i: (i, 0))],
        out_specs=pl.BlockSpec((_BLK, _D_OUT), lambda i: (i, 0)),
        out_shape=jax.ShapeDtypeStruct((_NP, _D_OUT), jnp.float32),
    )(acc2, r2, b2, ci)

    return out[:_N]


# speedup vs baseline: 1.1537x; 1.1537x over previous
import functools

import jax
import jax.numpy as jnp
from jax import lax
from jax.experimental import pallas as pl
from jax.experimental.pallas import tpu as pltpu
from jax.experimental.pallas import tpu_sc as plsc

_N = 10000
_NP = 10240
_E = 320000
_D_IN, _HID, _D_OUT = 128, 64, 32
_T = 128

_NC, _NS = 2, 16
_NW = _NC * _NS
_CH = 128
_K = 79
_EP = _NW * _K * _CH
_RPS = _NP // _NS

_BLK = 256
_HI = jax.lax.Precision.HIGHEST


def _mm1_body(x_ref, w_ref, tab_ref, r_ref):
    m = jnp.dot(x_ref[...], w_ref[...], preferred_element_type=jnp.float32,
                precision=_HI)
    lane = lax.broadcasted_iota(jnp.int32, (_BLK, _T - _HID), 1)
    ones = jnp.where(lane == 0, 1.0, 0.0).astype(jnp.float32)
    tab_ref[...] = jnp.concatenate([m[:, :_HID], ones], axis=1)
    r_ref[...] = m[:, _HID:]


def _mid_body(acc_ref, r1_ref, b1_ref, w2_ref, tab2_ref, r2_ref, ci_ref):
    acc = acc_ref[0] + acc_ref[1]
    cnt_inv = 1.0 / jnp.maximum(acc[:, _HID:_HID + 1], 1.0)
    h = jnp.maximum(acc[:, :_HID] * cnt_inv + b1_ref[0] + r1_ref[...], 0.0)
    m = jnp.dot(h, w2_ref[...], preferred_element_type=jnp.float32,
                precision=_HI)
    zeros = jnp.zeros((_BLK, _T - _D_OUT), jnp.float32)
    tab2_ref[...] = jnp.concatenate([m[:, :_D_OUT], zeros], axis=1)
    r2_ref[...] = m[:, _D_OUT:]
    ci_ref[...] = jnp.broadcast_to(cnt_inv, (_BLK, 8))


def _out_body(acc_ref, r2_ref, b2_ref, ci_ref, o_ref):
    acc = acc_ref[0, :, :_D_OUT] + acc_ref[1, :, :_D_OUT]
    o_ref[...] = jnp.maximum(acc * ci_ref[:, :1] + b2_ref[0] + r2_ref[...],
                             0.0)


def _sc_segsum(table, eidx, zeros):
    mesh = plsc.VectorSubcoreMesh(core_axis_name="c", subcore_axis_name="s")

    @functools.partial(
        pl.kernel, mesh=mesh,
        out_type=jax.ShapeDtypeStruct((_NC, _NP, _T), jnp.float32),
        scratch_types=[
            pltpu.VMEM_SHARED((_NP, _T), jnp.float32),
        ],
    )
    def k(tab_hbm, eidx_hbm, z_hbm, out_hbm, acc_sh):
        cid = lax.axis_index("c")
        sid = lax.axis_index("s")
        wid = sid * _NC + cid
        row0 = sid * _RPS
        pltpu.sync_copy(z_hbm.at[pl.ds(row0, _RPS)],
                        acc_sh.at[pl.ds(row0, _RPS)])
        plsc.subcore_barrier()

        def body(idx_v, buf):
            pltpu.sync_copy(eidx_hbm.at[wid], idx_v)

            @pl.loop(0, _K)
            def _(j):
                pltpu.sync_copy(tab_hbm.at[idx_v.at[j, 0]], buf)
                pltpu.sync_copy(buf, acc_sh.at[idx_v.at[j, 1]], add=True)

        pl.run_scoped(body,
                      pltpu.VMEM((_K, 2, _CH), jnp.int32),
                      pltpu.VMEM((_CH, _T), jnp.float32))
        plsc.subcore_barrier()
        pltpu.sync_copy(acc_sh.at[pl.ds(row0, _RPS)],
                        out_hbm.at[cid, pl.ds(row0, _RPS)])

    return k(table, eidx, zeros)


def kernel(x, edge_index, Wl1, bl1, Wr1, Wl2, bl2, Wr2):
    xp = jnp.pad(x, ((0, _NP - _N), (0, 0)))
    w1c = jnp.concatenate([Wl1.T, Wr1.T], axis=1)
    w2c = jnp.concatenate([Wl2.T, Wr2.T], axis=1)
    b1 = bl1.reshape(1, _HID)
    b2 = bl2.reshape(1, _D_OUT)
    pad_idx = (_N + jnp.arange(_EP - _E, dtype=jnp.int32) % (_NP - _N))
    pad_idx = jnp.broadcast_to(pad_idx, (2, _EP - _E))
    eidx = jnp.concatenate([edge_index, pad_idx], axis=1)
    eidx = eidx.reshape(2, _NW, _K, _CH)
    eidx = jnp.transpose(eidx, (1, 2, 0, 3))
    z = jnp.zeros((_NP, _T), jnp.float32)

    tab1, r1 = pl.pallas_call(
        _mm1_body,
        grid=(_NP // _BLK,),
        in_specs=[pl.BlockSpec((_BLK, _D_IN), lambda i: (i, 0)),
                  pl.BlockSpec((_D_IN, 2 * _HID), lambda i: (0, 0))],
        out_specs=[pl.BlockSpec((_BLK, _T), lambda i: (i, 0)),
                   pl.BlockSpec((_BLK, _HID), lambda i: (i, 0))],
        out_shape=[jax.ShapeDtypeStruct((_NP, _T), jnp.float32),
                   jax.ShapeDtypeStruct((_NP, _HID), jnp.float32)],
    )(xp, w1c)

    acc1 = _sc_segsum(tab1, eidx, z)

    tab2, r2, ci = pl.pallas_call(
        _mid_body,
        grid=(_NP // _BLK,),
        in_specs=[pl.BlockSpec((_NC, _BLK, _T), lambda i: (0, i, 0)),
                  pl.BlockSpec((_BLK, _HID), lambda i: (i, 0)),
                  pl.BlockSpec((1, _HID), lambda i: (0, 0)),
                  pl.BlockSpec((_HID, 2 * _D_OUT), lambda i: (0, 0))],
        out_specs=[pl.BlockSpec((_BLK, _T), lambda i: (i, 0)),
                   pl.BlockSpec((_BLK, _D_OUT), lambda i: (i, 0)),
                   pl.BlockSpec((_BLK, 8), lambda i: (i, 0))],
        out_shape=[jax.ShapeDtypeStruct((_NP, _T), jnp.float32),
                   jax.ShapeDtypeStruct((_NP, _D_OUT), jnp.float32),
                   jax.ShapeDtypeStruct((_NP, 8), jnp.float32)],
    )(acc1, r1, b1, w2c)

    acc2 = _sc_segsum(tab2, eidx, z)

    out = pl.pallas_call(
        _out_body,
        grid=(_NP // _BLK,),
        in_specs=[pl.BlockSpec((_NC, _BLK, _T), lambda i: (0, i, 0)),
                  pl.BlockSpec((_BLK, _D_OUT), lambda i: (i, 0)),
                  pl.BlockSpec((1, _D_OUT), lambda i: (0, 0)),
                  pl.BlockSpec((_BLK, 8), lambda i: (i, 0))],
        out_specs=pl.BlockSpec((_BLK, _D_OUT), lambda i: (i, 0)),
        out_shape=jax.ShapeDtypeStruct((_NP, _D_OUT), jnp.float32),
    )(acc2, r2, b2, ci)

    return out[:_N]


# speedup vs baseline: 1.1542x; 1.0005x over previous
import functools

import jax
import jax.numpy as jnp
from jax import lax
from jax.experimental import pallas as pl
from jax.experimental.pallas import tpu as pltpu
from jax.experimental.pallas import tpu_sc as plsc

_N = 10000
_NP = 10240
_E = 320000
_D_IN, _HID, _D_OUT = 128, 64, 32
_T = 128

_NC, _NS = 2, 16
_NW = _NC * _NS
_CH = 128
_K = 79
_EP = _NW * _K * _CH
_RPS = _NP // _NS

_BLK = 256
_HI = jax.lax.Precision.HIGHEST


def _mm1_body(x_ref, w_ref, tab_ref, r_ref):
    m = jnp.dot(x_ref[...], w_ref[...], preferred_element_type=jnp.float32,
                precision=_HI)
    lane = lax.broadcasted_iota(jnp.int32, (_BLK, _T - _HID), 1)
    ones = jnp.where(lane == 0, 1.0, 0.0).astype(jnp.float32)
    tab_ref[...] = jnp.concatenate([m[:, :_HID], ones], axis=1)
    r_ref[...] = m[:, _HID:]


def _mid_body(acc_ref, r1_ref, b1_ref, w2_ref, tab2_ref, r2_ref, ci_ref):
    acc = acc_ref[0] + acc_ref[1]
    cnt_inv = 1.0 / jnp.maximum(acc[:, _HID:_HID + 1], 1.0)
    h = jnp.maximum(acc[:, :_HID] * cnt_inv + b1_ref[0] + r1_ref[...], 0.0)
    m = jnp.dot(h, w2_ref[...], preferred_element_type=jnp.float32,
                precision=_HI)
    zeros = jnp.zeros((_BLK, _T - _D_OUT), jnp.float32)
    tab2_ref[...] = jnp.concatenate([m[:, :_D_OUT], zeros], axis=1)
    r2_ref[...] = m[:, _D_OUT:]
    ci_ref[...] = jnp.broadcast_to(cnt_inv, (_BLK, 8))


def _out_body(acc_ref, r2_ref, b2_ref, ci_ref, o_ref):
    acc = acc_ref[0, :, :_D_OUT] + acc_ref[1, :, :_D_OUT]
    o_ref[...] = jnp.maximum(acc * ci_ref[:, :1] + b2_ref[0] + r2_ref[...],
                             0.0)


def _sc_segsum(table, eidx, zeros):
    mesh = plsc.VectorSubcoreMesh(core_axis_name="c", subcore_axis_name="s")

    @functools.partial(
        pl.kernel, mesh=mesh,
        out_type=jax.ShapeDtypeStruct((_NC, _NP, _T), jnp.float32),
        scratch_types=[
            pltpu.VMEM_SHARED((_NP, _T), jnp.float32),
        ],
    )
    def k(tab_hbm, eidx_hbm, z_hbm, out_hbm, acc_sh):
        cid = lax.axis_index("c")
        sid = lax.axis_index("s")
        wid = sid * _NC + cid
        row0 = sid * _RPS
        pltpu.sync_copy(z_hbm.at[pl.ds(row0, _RPS)],
                        acc_sh.at[pl.ds(row0, _RPS)])
        plsc.subcore_barrier()

        def body(idx_v, buf):
            pltpu.sync_copy(eidx_hbm.at[wid], idx_v)

            @pl.loop(0, _K)
            def _(j):
                pltpu.sync_copy(tab_hbm.at[idx_v.at[j, 0]], buf)
                pltpu.sync_copy(buf, acc_sh.at[idx_v.at[j, 1]], add=True)

        pl.run_scoped(body,
                      pltpu.VMEM((_K, 2, _CH), jnp.int32),
                      pltpu.VMEM((_CH, _T), jnp.float32))
        plsc.subcore_barrier()
        pltpu.sync_copy(acc_sh.at[pl.ds(row0, _RPS)],
                        out_hbm.at[cid, pl.ds(row0, _RPS)])

    return k(table, eidx, zeros)


def _sc_segsum_staged(table, eidx, zeros, d):
    mesh = plsc.VectorSubcoreMesh(core_axis_name="c", subcore_axis_name="s")

    @functools.partial(
        pl.kernel, mesh=mesh,
        out_type=jax.ShapeDtypeStruct((_NC, _NP, d), jnp.float32),
        scratch_types=[
            pltpu.VMEM_SHARED((_NP, d), jnp.float32),
            pltpu.VMEM_SHARED((_NP, d), jnp.float32),
        ],
    )
    def k(tab_hbm, eidx_hbm, z_hbm, out_hbm, acc_sh, tab_sh):
        cid = lax.axis_index("c")
        sid = lax.axis_index("s")
        wid = sid * _NC + cid
        row0 = sid * _RPS
        pltpu.sync_copy(z_hbm.at[pl.ds(row0, _RPS)],
                        acc_sh.at[pl.ds(row0, _RPS)])
        pltpu.sync_copy(tab_hbm.at[pl.ds(row0, _RPS)],
                        tab_sh.at[pl.ds(row0, _RPS)])
        plsc.subcore_barrier()

        def body(idx_v, buf):
            pltpu.sync_copy(eidx_hbm.at[wid], idx_v)

            @pl.loop(0, _K)
            def _(j):
                pltpu.sync_copy(tab_sh.at[idx_v.at[j, 0]], buf)
                pltpu.sync_copy(buf, acc_sh.at[idx_v.at[j, 1]], add=True)

        pl.run_scoped(body,
                      pltpu.VMEM((_K, 2, _CH), jnp.int32),
                      pltpu.VMEM((_CH, d), jnp.float32))
        plsc.subcore_barrier()
        pltpu.sync_copy(acc_sh.at[pl.ds(row0, _RPS)],
                        out_hbm.at[cid, pl.ds(row0, _RPS)])

    return k(table, eidx, zeros)


def kernel(x, edge_index, Wl1, bl1, Wr1, Wl2, bl2, Wr2):
    xp = jnp.pad(x, ((0, _NP - _N), (0, 0)))
    w1c = jnp.concatenate([Wl1.T, Wr1.T], axis=1)
    w2c = jnp.concatenate([Wl2.T, Wr2.T], axis=1)
    b1 = bl1.reshape(1, _HID)
    b2 = bl2.reshape(1, _D_OUT)
    pad_idx = (_N + jnp.arange(_EP - _E, dtype=jnp.int32) % (_NP - _N))
    pad_idx = jnp.broadcast_to(pad_idx, (2, _EP - _E))
    eidx = jnp.concatenate([edge_index, pad_idx], axis=1)
    eidx = eidx.reshape(2, _NW, _K, _CH)
    eidx = jnp.transpose(eidx, (1, 2, 0, 3))
    z = jnp.zeros((_NP, _T), jnp.float32)

    tab1, r1 = pl.pallas_call(
        _mm1_body,
        grid=(_NP // _BLK,),
        in_specs=[pl.BlockSpec((_BLK, _D_IN), lambda i: (i, 0)),
                  pl.BlockSpec((_D_IN, 2 * _HID), lambda i: (0, 0))],
        out_specs=[pl.BlockSpec((_BLK, _T), lambda i: (i, 0)),
                   pl.BlockSpec((_BLK, _HID), lambda i: (i, 0))],
        out_shape=[jax.ShapeDtypeStruct((_NP, _T), jnp.float32),
                   jax.ShapeDtypeStruct((_NP, _HID), jnp.float32)],
    )(xp, w1c)

    acc1 = _sc_segsum(tab1, eidx, z)

    tab2, r2, ci = pl.pallas_call(
        _mid_body,
        grid=(_NP // _BLK,),
        in_specs=[pl.BlockSpec((_NC, _BLK, _T), lambda i: (0, i, 0)),
                  pl.BlockSpec((_BLK, _HID), lambda i: (i, 0)),
                  pl.BlockSpec((1, _HID), lambda i: (0, 0)),
                  pl.BlockSpec((_HID, 2 * _D_OUT), lambda i: (0, 0))],
        out_specs=[pl.BlockSpec((_BLK, _T), lambda i: (i, 0)),
                   pl.BlockSpec((_BLK, _D_OUT), lambda i: (i, 0)),
                   pl.BlockSpec((_BLK, 8), lambda i: (i, 0))],
        out_shape=[jax.ShapeDtypeStruct((_NP, _T), jnp.float32),
                   jax.ShapeDtypeStruct((_NP, _D_OUT), jnp.float32),
                   jax.ShapeDtypeStruct((_NP, 8), jnp.float32)],
    )(acc1, r1, b1, w2c)

    acc2 = _sc_segsum(tab2, eidx, z)

    out = pl.pallas_call(
        _out_body,
        grid=(_NP // _BLK,),
        in_specs=[pl.BlockSpec((_NC, _BLK, _T), lambda i: (0, i, 0)),
                  pl.BlockSpec((_BLK, _D_OUT), lambda i: (i, 0)),
                  pl.BlockSpec((1, _D_OUT), lambda i: (0, 0)),
                  pl.BlockSpec((_BLK, 8), lambda i: (i, 0))],
        out_specs=pl.BlockSpec((_BLK, _D_OUT), lambda i: (i, 0)),
        out_shape=jax.ShapeDtypeStruct((_NP, _D_OUT), jnp.float32),
    )(acc2, r2, b2, ci)

    return out[:_N]


# speedup vs baseline: 1.4291x; 1.2381x over previous
import functools

import jax
import jax.numpy as jnp
from jax import lax
from jax.experimental import pallas as pl
from jax.experimental.pallas import tpu as pltpu
from jax.experimental.pallas import tpu_sc as plsc

_N = 10000
_NP = 10240
_E = 320000
_D_IN, _HID, _D_OUT = 128, 64, 32
_T = 128

_NC, _NS = 2, 16
_NW = _NC * _NS
_CH = 64
_K = 158
_KP = 79
_CHW = 128
_EP = _NW * _K * _CH
_RPS = _NP // _NS

_BLK = 256
_HI = jax.lax.Precision.HIGHEST


def _mm1_body(x_ref, w_ref, tab_ref, r_ref):
    m = jnp.dot(x_ref[...], w_ref[...], preferred_element_type=jnp.float32,
                precision=_HI)
    lane = lax.broadcasted_iota(jnp.int32, (_BLK, _T - _HID), 1)
    ones = jnp.where(lane == 0, 1.0, 0.0).astype(jnp.float32)
    tab_ref[...] = jnp.concatenate([m[:, :_HID], ones], axis=1)
    r_ref[...] = m[:, _HID:]


def _mid_body(acc_ref, r1_ref, b1_ref, w2_ref, tab2_ref, r2_ref, ci_ref):
    acc = acc_ref[0] + acc_ref[1]
    cnt_inv = 1.0 / jnp.maximum(acc[:, _HID:_HID + 1], 1.0)
    h = jnp.maximum(acc[:, :_HID] * cnt_inv + b1_ref[0] + r1_ref[...], 0.0)
    m = jnp.dot(h, w2_ref[...], preferred_element_type=jnp.float32,
                precision=_HI)
    zeros = jnp.zeros((_BLK, _T - _D_OUT), jnp.float32)
    tab2_ref[...] = jnp.concatenate([m[:, :_D_OUT], zeros], axis=1)
    r2_ref[...] = m[:, _D_OUT:]
    ci_ref[...] = jnp.broadcast_to(cnt_inv, (_BLK, 8))


def _out_body(acc_ref, r2_ref, b2_ref, ci_ref, o_ref):
    acc = acc_ref[0, :, :_D_OUT] + acc_ref[1, :, :_D_OUT]
    o_ref[...] = jnp.maximum(acc * ci_ref[:, :1] + b2_ref[0] + r2_ref[...],
                             0.0)


def _sc_segsum(table, eidx, zeros):
    mesh = plsc.VectorSubcoreMesh(core_axis_name="c", subcore_axis_name="s")

    @functools.partial(
        pl.kernel, mesh=mesh,
        out_type=jax.ShapeDtypeStruct((_NC, _NP, _T), jnp.float32),
        scratch_types=[
            pltpu.VMEM_SHARED((_NP, _T), jnp.float32),
            pltpu.SemaphoreType.DMA,
            pltpu.SemaphoreType.DMA,
        ],
    )
    def k(tab_hbm, eidx_hbm, z_hbm, out_hbm, acc_sh, sem0, sem1):
        cid = lax.axis_index("c")
        sid = lax.axis_index("s")
        wid = sid * _NC + cid
        row0 = sid * _RPS

        pltpu.sync_copy(z_hbm.at[pl.ds(row0, _RPS)],
                        acc_sh.at[pl.ds(row0, _RPS)])
        plsc.subcore_barrier()

        def body(idx_v, buf):
            pltpu.sync_copy(eidx_hbm.at[wid], idx_v)

            def src(j):
                return idx_v.at[j // 2, 0, pl.ds((j % 2) * _CH, _CH)]

            def dst(j):
                return idx_v.at[j // 2, 1, pl.ds((j % 2) * _CH, _CH)]

            pltpu.async_copy(tab_hbm.at[src(0)], buf.at[0], sem0)
            pltpu.async_copy(tab_hbm.at[src(1)], buf.at[1], sem1)

            @pl.loop(0, _K)
            def _(j):
                b = j & 1

                @pl.when(b == 0)
                def _():
                    pltpu.make_async_copy(tab_hbm.at[src(j)],
                                          buf.at[0], sem0).wait()

                @pl.when(b == 1)
                def _():
                    pltpu.make_async_copy(tab_hbm.at[src(j)],
                                          buf.at[1], sem1).wait()

                pltpu.sync_copy(buf.at[b], acc_sh.at[dst(j)], add=True)

                @pl.when((j + 2 < _K) & (b == 0))
                def _():
                    pltpu.async_copy(tab_hbm.at[src(j + 2)], buf.at[0], sem0)

                @pl.when((j + 2 < _K) & (b == 1))
                def _():
                    pltpu.async_copy(tab_hbm.at[src(j + 2)], buf.at[1], sem1)

        pl.run_scoped(body,
                      pltpu.VMEM((_KP, 2, _CHW), jnp.int32),
                      pltpu.VMEM((2, _CH, _T), jnp.float32))
        plsc.subcore_barrier()
        pltpu.sync_copy(acc_sh.at[pl.ds(row0, _RPS)],
                        out_hbm.at[cid, pl.ds(row0, _RPS)])

    return k(table, eidx, zeros)


def _sc_segsum_staged(table, eidx, zeros, d):
    mesh = plsc.VectorSubcoreMesh(core_axis_name="c", subcore_axis_name="s")

    @functools.partial(
        pl.kernel, mesh=mesh,
        out_type=jax.ShapeDtypeStruct((_NC, _NP, d), jnp.float32),
        scratch_types=[
            pltpu.VMEM_SHARED((_NP, d), jnp.float32),
            pltpu.VMEM_SHARED((_NP, d), jnp.float32),
        ],
    )
    def k(tab_hbm, eidx_hbm, z_hbm, out_hbm, acc_sh, tab_sh):
        cid = lax.axis_index("c")
        sid = lax.axis_index("s")
        wid = sid * _NC + cid
        row0 = sid * _RPS
        pltpu.sync_copy(z_hbm.at[pl.ds(row0, _RPS)],
                        acc_sh.at[pl.ds(row0, _RPS)])
        pltpu.sync_copy(tab_hbm.at[pl.ds(row0, _RPS)],
                        tab_sh.at[pl.ds(row0, _RPS)])
        plsc.subcore_barrier()

        def body(idx_v, buf):
            pltpu.sync_copy(eidx_hbm.at[wid], idx_v)

            @pl.loop(0, _K)
            def _(j):
                pltpu.sync_copy(tab_sh.at[idx_v.at[j, 0]], buf)
                pltpu.sync_copy(buf, acc_sh.at[idx_v.at[j, 1]], add=True)

        pl.run_scoped(body,
                      pltpu.VMEM((_K, 2, _CH), jnp.int32),
                      pltpu.VMEM((_CH, d), jnp.float32))
        plsc.subcore_barrier()
        pltpu.sync_copy(acc_sh.at[pl.ds(row0, _RPS)],
                        out_hbm.at[cid, pl.ds(row0, _RPS)])

    return k(table, eidx, zeros)


def kernel(x, edge_index, Wl1, bl1, Wr1, Wl2, bl2, Wr2):
    xp = jnp.pad(x, ((0, _NP - _N), (0, 0)))
    w1c = jnp.concatenate([Wl1.T, Wr1.T], axis=1)
    w2c = jnp.concatenate([Wl2.T, Wr2.T], axis=1)
    b1 = bl1.reshape(1, _HID)
    b2 = bl2.reshape(1, _D_OUT)
    pad_idx = (_N + jnp.arange(_EP - _E, dtype=jnp.int32) % (_NP - _N))
    pad_idx = jnp.broadcast_to(pad_idx, (2, _EP - _E))
    eidx = jnp.concatenate([edge_index, pad_idx], axis=1)
    eidx = eidx.reshape(2, _NW, _KP, _CHW)
    eidx = jnp.transpose(eidx, (1, 2, 0, 3))
    z = jnp.zeros((_NP, _T), jnp.float32)

    tab1, r1 = pl.pallas_call(
        _mm1_body,
        grid=(_NP // _BLK,),
        in_specs=[pl.BlockSpec((_BLK, _D_IN), lambda i: (i, 0)),
                  pl.BlockSpec((_D_IN, 2 * _HID), lambda i: (0, 0))],
        out_specs=[pl.BlockSpec((_BLK, _T), lambda i: (i, 0)),
                   pl.BlockSpec((_BLK, _HID), lambda i: (i, 0))],
        out_shape=[jax.ShapeDtypeStruct((_NP, _T), jnp.float32),
                   jax.ShapeDtypeStruct((_NP, _HID), jnp.float32)],
    )(xp, w1c)

    acc1 = _sc_segsum(tab1, eidx, z)

    tab2, r2, ci = pl.pallas_call(
        _mid_body,
        grid=(_NP // _BLK,),
        in_specs=[pl.BlockSpec((_NC, _BLK, _T), lambda i: (0, i, 0)),
                  pl.BlockSpec((_BLK, _HID), lambda i: (i, 0)),
                  pl.BlockSpec((1, _HID), lambda i: (0, 0)),
                  pl.BlockSpec((_HID, 2 * _D_OUT), lambda i: (0, 0))],
        out_specs=[pl.BlockSpec((_BLK, _T), lambda i: (i, 0)),
                   pl.BlockSpec((_BLK, _D_OUT), lambda i: (i, 0)),
                   pl.BlockSpec((_BLK, 8), lambda i: (i, 0))],
        out_shape=[jax.ShapeDtypeStruct((_NP, _T), jnp.float32),
                   jax.ShapeDtypeStruct((_NP, _D_OUT), jnp.float32),
                   jax.ShapeDtypeStruct((_NP, 8), jnp.float32)],
    )(acc1, r1, b1, w2c)

    acc2 = _sc_segsum(tab2, eidx, z)

    out = pl.pallas_call(
        _out_body,
        grid=(_NP // _BLK,),
        in_specs=[pl.BlockSpec((_NC, _BLK, _T), lambda i: (0, i, 0)),
                  pl.BlockSpec((_BLK, _D_OUT), lambda i: (i, 0)),
                  pl.BlockSpec((1, _D_OUT), lambda i: (0, 0)),
                  pl.BlockSpec((_BLK, 8), lambda i: (i, 0))],
        out_specs=pl.BlockSpec((_BLK, _D_OUT), lambda i: (i, 0)),
        out_shape=jax.ShapeDtypeStruct((_NP, _D_OUT), jnp.float32),
    )(acc2, r2, b2, ci)

    return out[:_N]


# speedup vs baseline: 1.6810x; 1.1763x over previous
import functools

import jax
import jax.numpy as jnp
from jax import lax
from jax.experimental import pallas as pl
from jax.experimental.pallas import tpu as pltpu
from jax.experimental.pallas import tpu_sc as plsc

_N = 10000
_NP = 10240
_E = 320000
_D_IN, _HID, _D_OUT = 128, 64, 32
_T = 128

_NC, _NS = 2, 16
_NW = _NC * _NS
_CH = 64
_K = 158
_KP = 79
_CHW = 128
_EP = _NW * _K * _CH
_RPS = _NP // _NS

_BLK = 256
_HI = jax.lax.Precision.HIGHEST


def _mm1_body(x_ref, w_ref, tab_ref, r_ref):
    m = jnp.dot(x_ref[...], w_ref[...], preferred_element_type=jnp.float32,
                precision=_HI)
    lane = lax.broadcasted_iota(jnp.int32, (_BLK, _T - _HID), 1)
    ones = jnp.where(lane == 0, 1.0, 0.0).astype(jnp.float32)
    tab_ref[...] = jnp.concatenate([m[:, :_HID], ones], axis=1)
    r_ref[...] = m[:, _HID:]


def _mid_body(acc_ref, r1_ref, b1_ref, w2_ref, tab2_ref, r2_ref, ci_ref):
    acc = acc_ref[0] + acc_ref[1]
    cnt_inv = 1.0 / jnp.maximum(acc[:, _HID:_HID + 1], 1.0)
    h = jnp.maximum(acc[:, :_HID] * cnt_inv + b1_ref[0] + r1_ref[...], 0.0)
    m = jnp.dot(h, w2_ref[...], preferred_element_type=jnp.float32,
                precision=_HI)
    zeros = jnp.zeros((_BLK, _T - _D_OUT), jnp.float32)
    tab2_ref[...] = jnp.concatenate([m[:, :_D_OUT], zeros], axis=1)
    r2_ref[...] = m[:, _D_OUT:]
    ci_ref[...] = jnp.broadcast_to(cnt_inv, (_BLK, 8))


def _out_body(acc_ref, r2_ref, b2_ref, ci_ref, o_ref):
    acc = acc_ref[0, :, :_D_OUT] + acc_ref[1, :, :_D_OUT]
    o_ref[...] = jnp.maximum(acc * ci_ref[:, :1] + b2_ref[0] + r2_ref[...],
                             0.0)


def _sc_segsum(table, eidx, zeros):
    mesh = plsc.VectorSubcoreMesh(core_axis_name="c", subcore_axis_name="s")

    @functools.partial(
        pl.kernel, mesh=mesh,
        out_type=jax.ShapeDtypeStruct((_NC, _NP, _T), jnp.float32),
        scratch_types=[
            pltpu.VMEM_SHARED((_NP, _T), jnp.float32),
            pltpu.SemaphoreType.DMA,
            pltpu.SemaphoreType.DMA,
            pltpu.SemaphoreType.DMA,
        ],
    )
    def k(tab_hbm, eidx_hbm, z_hbm, out_hbm, acc_sh, sem0, sem1, sem2):
        cid = lax.axis_index("c")
        sid = lax.axis_index("s")
        wid = sid * _NC + cid
        row0 = sid * _RPS

        pltpu.sync_copy(z_hbm.at[pl.ds(row0, _RPS)],
                        acc_sh.at[pl.ds(row0, _RPS)])
        plsc.subcore_barrier()

        def body(idx_v, buf):
            pltpu.sync_copy(eidx_hbm.at[wid], idx_v)

            def src(j):
                return idx_v.at[j // 2, 0, pl.ds((j % 2) * _CH, _CH)]

            def dst(j):
                return idx_v.at[j // 2, 1, pl.ds((j % 2) * _CH, _CH)]

            pltpu.async_copy(tab_hbm.at[src(0)], buf.at[0], sem0)
            pltpu.async_copy(tab_hbm.at[src(1)], buf.at[1], sem1)
            pltpu.async_copy(tab_hbm.at[src(2)], buf.at[2], sem2)

            @pl.loop(0, _K)
            def _(j):
                b = j % 3

                @pl.when(b == 0)
                def _():
                    pltpu.make_async_copy(tab_hbm.at[src(j)],
                                          buf.at[0], sem0).wait()

                @pl.when(b == 1)
                def _():
                    pltpu.make_async_copy(tab_hbm.at[src(j)],
                                          buf.at[1], sem1).wait()

                @pl.when(b == 2)
                def _():
                    pltpu.make_async_copy(tab_hbm.at[src(j)],
                                          buf.at[2], sem2).wait()

                pltpu.sync_copy(buf.at[b], acc_sh.at[dst(j)], add=True)

                @pl.when((j + 3 < _K) & (b == 0))
                def _():
                    pltpu.async_copy(tab_hbm.at[src(j + 3)], buf.at[0], sem0)

                @pl.when((j + 3 < _K) & (b == 1))
                def _():
                    pltpu.async_copy(tab_hbm.at[src(j + 3)], buf.at[1], sem1)

                @pl.when((j + 3 < _K) & (b == 2))
                def _():
                    pltpu.async_copy(tab_hbm.at[src(j + 3)], buf.at[2], sem2)

        pl.run_scoped(body,
                      pltpu.VMEM((_KP, 2, _CHW), jnp.int32),
                      pltpu.VMEM((3, _CH, _T), jnp.float32))
        plsc.subcore_barrier()
        pltpu.sync_copy(acc_sh.at[pl.ds(row0, _RPS)],
                        out_hbm.at[cid, pl.ds(row0, _RPS)])

    return k(table, eidx, zeros)


def _sc_segsum_staged(table, eidx, zeros, d):
    mesh = plsc.VectorSubcoreMesh(core_axis_name="c", subcore_axis_name="s")

    @functools.partial(
        pl.kernel, mesh=mesh,
        out_type=jax.ShapeDtypeStruct((_NC, _NP, d), jnp.float32),
        scratch_types=[
            pltpu.VMEM_SHARED((_NP, d), jnp.float32),
            pltpu.VMEM_SHARED((_NP, d), jnp.float32),
        ],
    )
    def k(tab_hbm, eidx_hbm, z_hbm, out_hbm, acc_sh, tab_sh):
        cid = lax.axis_index("c")
        sid = lax.axis_index("s")
        wid = sid * _NC + cid
        row0 = sid * _RPS
        pltpu.sync_copy(z_hbm.at[pl.ds(row0, _RPS)],
                        acc_sh.at[pl.ds(row0, _RPS)])
        pltpu.sync_copy(tab_hbm.at[pl.ds(row0, _RPS)],
                        tab_sh.at[pl.ds(row0, _RPS)])
        plsc.subcore_barrier()

        def body(idx_v, buf):
            pltpu.sync_copy(eidx_hbm.at[wid], idx_v)

            @pl.loop(0, _K)
            def _(j):
                pltpu.sync_copy(tab_sh.at[idx_v.at[j, 0]], buf)
                pltpu.sync_copy(buf, acc_sh.at[idx_v.at[j, 1]], add=True)

        pl.run_scoped(body,
                      pltpu.VMEM((_K, 2, _CH), jnp.int32),
                      pltpu.VMEM((_CH, d), jnp.float32))
        plsc.subcore_barrier()
        pltpu.sync_copy(acc_sh.at[pl.ds(row0, _RPS)],
                        out_hbm.at[cid, pl.ds(row0, _RPS)])

    return k(table, eidx, zeros)


def kernel(x, edge_index, Wl1, bl1, Wr1, Wl2, bl2, Wr2):
    xp = jnp.pad(x, ((0, _NP - _N), (0, 0)))
    w1c = jnp.concatenate([Wl1.T, Wr1.T], axis=1)
    w2c = jnp.concatenate([Wl2.T, Wr2.T], axis=1)
    b1 = bl1.reshape(1, _HID)
    b2 = bl2.reshape(1, _D_OUT)
    pad_idx = (_N + jnp.arange(_EP - _E, dtype=jnp.int32) % (_NP - _N))
    pad_idx = jnp.broadcast_to(pad_idx, (2, _EP - _E))
    eidx = jnp.concatenate([edge_index, pad_idx], axis=1)
    eidx = eidx.reshape(2, _NW, _KP, _CHW)
    eidx = jnp.transpose(eidx, (1, 2, 0, 3))
    z = jnp.zeros((_NP, _T), jnp.float32)

    tab1, r1 = pl.pallas_call(
        _mm1_body,
        grid=(_NP // _BLK,),
        in_specs=[pl.BlockSpec((_BLK, _D_IN), lambda i: (i, 0)),
                  pl.BlockSpec((_D_IN, 2 * _HID), lambda i: (0, 0))],
        out_specs=[pl.BlockSpec((_BLK, _T), lambda i: (i, 0)),
                   pl.BlockSpec((_BLK, _HID), lambda i: (i, 0))],
        out_shape=[jax.ShapeDtypeStruct((_NP, _T), jnp.float32),
                   jax.ShapeDtypeStruct((_NP, _HID), jnp.float32)],
    )(xp, w1c)

    acc1 = _sc_segsum(tab1, eidx, z)

    tab2, r2, ci = pl.pallas_call(
        _mid_body,
        grid=(_NP // _BLK,),
        in_specs=[pl.BlockSpec((_NC, _BLK, _T), lambda i: (0, i, 0)),
                  pl.BlockSpec((_BLK, _HID), lambda i: (i, 0)),
                  pl.BlockSpec((1, _HID), lambda i: (0, 0)),
                  pl.BlockSpec((_HID, 2 * _D_OUT), lambda i: (0, 0))],
        out_specs=[pl.BlockSpec((_BLK, _T), lambda i: (i, 0)),
                   pl.BlockSpec((_BLK, _D_OUT), lambda i: (i, 0)),
                   pl.BlockSpec((_BLK, 8), lambda i: (i, 0))],
        out_shape=[jax.ShapeDtypeStruct((_NP, _T), jnp.float32),
                   jax.ShapeDtypeStruct((_NP, _D_OUT), jnp.float32),
                   jax.ShapeDtypeStruct((_NP, 8), jnp.float32)],
    )(acc1, r1, b1, w2c)

    acc2 = _sc_segsum(tab2, eidx, z)

    out = pl.pallas_call(
        _out_body,
        grid=(_NP // _BLK,),
        in_specs=[pl.BlockSpec((_NC, _BLK, _T), lambda i: (0, i, 0)),
                  pl.BlockSpec((_BLK, _D_OUT), lambda i: (i, 0)),
                  pl.BlockSpec((1, _D_OUT), lambda i: (0, 0)),
                  pl.BlockSpec((_BLK, 8), lambda i: (i, 0))],
        out_specs=pl.BlockSpec((_BLK, _D_OUT), lambda i: (i, 0)),
        out_shape=jax.ShapeDtypeStruct((_NP, _D_OUT), jnp.float32),
    )(acc2, r2, b2, ci)

    return out[:_N]
